# Initial kernel scaffold; baseline (speedup 1.0000x reference)
import jax
import jax.numpy as jnp
from jax.experimental import pallas as pl


def kernel(images, image_lengths, W_fc, b_fc, W1, b1, W2, b2):
    raise NotImplementedError("write your pallas kernel here")


# speedup vs baseline: 4.3769x; 4.3769x over previous
import functools

import numpy as np
import jax
import jax.numpy as jnp
from jax.experimental import pallas as pl
from jax.experimental.pallas import tpu as pltpu


NEG = -1e30


def _sinusoidal_pe(length, d):
    pos = np.arange(length, dtype=np.float32)[:, None]
    i = np.arange(d, dtype=np.float32)[None, :]
    angle = pos / np.power(10000.0, (2.0 * np.floor(i / 2.0)) / float(d))
    pe = np.where((np.arange(d)[None, :] % 2) == 0, np.sin(angle), np.cos(angle))
    return pe.astype(np.float32)


def _proj_kernel(img_ref, w_ref, b_ref, out_ref):
    x = img_ref[0]
    f = jax.lax.dot_general(x, w_ref[...], (((1,), (0,)), ((), ())),
                            preferred_element_type=jnp.float32)
    f = f + b_ref[...]
    n = jnp.sqrt(jnp.sum(f * f, axis=1, keepdims=True)) + 1e-8
    out_ref[0] = f / n


def _merge_kernel(ilen_ref, src_ref, dst_ref, val_ref, *, t1, r):
    b = pl.program_id(0)
    src = src_ref[0]
    dst = dst_ref[0]
    S = jax.lax.dot_general(src, dst, (((1,), (1,)), ((), ())),
                            preferred_element_type=jnp.float32)
    nm = jnp.max(S, axis=1, keepdims=True)
    jj = jax.lax.broadcasted_iota(jnp.int32, (t1, t1), 1)
    ii = jax.lax.broadcasted_iota(jnp.int32, (t1, t1), 0)
    nidx = jnp.min(jnp.where(S == nm, jj, t1), axis=1, keepdims=True)
    nm_row = jnp.transpose(nm)
    cmp = (nm_row > nm) | ((nm_row == nm) & (jj < ii))
    rank = jnp.sum(cmp.astype(jnp.float32), axis=1, keepdims=True)
    merged = rank < float(r)
    M = jnp.where(merged & (nidx == jj), 1.0, 0.0)
    adds = jax.lax.dot_general(M, src, (((0,), (0,)), ((), ())),
                               preferred_element_type=jnp.float32)
    cnt = jax.lax.dot_general(M, jnp.ones((t1, 1), jnp.float32),
                              (((0,), (0,)), ((), ())),
                              preferred_element_type=jnp.float32)
    dst2 = (dst + adds) / (1.0 + cnt)
    vd = ilen_ref[b] - t1
    icol = jax.lax.broadcasted_iota(jnp.int32, (t1, 1), 0)
    val_src = jnp.where(merged, NEG, src)
    val_dst = jnp.where(icol < vd, dst2, NEG)
    val_ref[0] = jnp.concatenate([val_src, val_dst], axis=0)


def _pool_kernel(ilen_ref, val_ref, pe_ref, w1_ref, b1_ref, w2_ref, out_ref, *, n, r):
    b = pl.program_id(0)
    x = val_ref[0]
    ct = x.shape[1]
    i = jax.lax.broadcasted_iota(jnp.int32, (n, 1), 0)
    k = 2
    while k <= n:
        j = k // 2
        while j >= 1:
            if j >= 8:
                a2j = n // (2 * j)
                x4 = x.reshape(a2j, 2, j, ct)
                lo = x4[:, 0]
                hi = x4[:, 1]
                mn = jnp.minimum(lo, hi)
                mx = jnp.maximum(lo, hi)
                a = jax.lax.broadcasted_iota(jnp.int32, (a2j, 1, 1), 0)
                desc = ((a * (2 * j)) & k) == 0
                nlo = jnp.where(desc, mx, mn)
                nhi = jnp.where(desc, mn, mx)
                x = jnp.concatenate([nlo[:, None], nhi[:, None]],
                                    axis=1).reshape(n, ct)
            else:
                pu = jnp.roll(x, j, axis=0)
                pd = jnp.roll(x, -j, axis=0)
                partner = jnp.where((i & j) != 0, pu, pd)
                is_lo = (i & j) == 0
                pair_desc = (i & k) == 0
                take_max = is_lo == pair_desc
                x = jnp.where(take_max, jnp.maximum(x, partner),
                              jnp.minimum(x, partner))
            j //= 2
        k *= 2
    h = jnp.tanh(jax.lax.dot_general(pe_ref[...], w1_ref[...],
                                     (((1,), (0,)), ((), ())),
                                     preferred_element_type=jnp.float32)
                 + b1_ref[...])
    lg = jax.lax.dot_general(h, w2_ref[...], (((1,), (0,)), ((), ())),
                             preferred_element_type=jnp.float32)[:, 0:1]
    lengths = ilen_ref[b] - r
    lg = jnp.where(i < lengths, lg, -1e9)
    m = jnp.max(lg)
    e = jnp.exp(lg - m)
    w = e / jnp.sum(e)
    pooled = jnp.sum(x * w, axis=0, keepdims=True)
    out_ref[0] = pooled


def _norm_kernel(p_ref, out_ref):
    p = p_ref[...]
    nrm = jnp.sqrt(jnp.sum(p * p, axis=2, keepdims=True)) + 1e-8
    out_ref[...] = p / nrm


@jax.jit
def kernel(images, image_lengths, W_fc, b_fc, W1, b1, W2, b2):
    B, T, K = images.shape
    C = W_fc.shape[0]
    t1 = T // 2
    r = min(T // 20, t1 // 2)
    L = T - r
    d = W1.shape[0]

    RT = min(256, T)
    feats_n = pl.pallas_call(
        _proj_kernel,
        grid=(B, T // RT),
        in_specs=[
            pl.BlockSpec((1, RT, K), lambda b, t: (b, t, 0)),
            pl.BlockSpec((K, C), lambda b, t: (0, 0)),
            pl.BlockSpec((1, C), lambda b, t: (0, 0)),
        ],
        out_specs=pl.BlockSpec((1, RT, C), lambda b, t: (b, t, 0)),
        out_shape=jax.ShapeDtypeStruct((B, T, C), jnp.float32),
    )(images, W_fc.T, b_fc.reshape(1, C))

    srcN = feats_n[:, ::2]
    dstN = feats_n[:, 1::2]

    val = pl.pallas_call(
        functools.partial(_merge_kernel, t1=t1, r=r),
        grid_spec=pltpu.PrefetchScalarGridSpec(
            num_scalar_prefetch=1,
            grid=(B,),
            in_specs=[
                pl.BlockSpec((1, t1, C), lambda b, s: (b, 0, 0)),
                pl.BlockSpec((1, t1, C), lambda b, s: (b, 0, 0)),
            ],
            out_specs=pl.BlockSpec((1, T, C), lambda b, s: (b, 0, 0)),
        ),
        out_shape=jax.ShapeDtypeStruct((B, T, C), jnp.float32),
    )(image_lengths, srcN, dstN)

    DP = 128
    pe = np.zeros((T, DP), np.float32)
    pe[:L, :d] = _sinusoidal_pe(L, d)
    W1p = np.zeros((DP, DP), np.float32)
    b1p = np.zeros((1, DP), np.float32)
    W2p = np.zeros((DP, DP), np.float32)
    pe = jnp.asarray(pe)
    W1p = jnp.asarray(W1p).at[:d, :d].set(W1)
    b1p = jnp.asarray(b1p).at[0, :d].set(b1)
    W2p = jnp.asarray(W2p).at[:d, 0].set(W2[:, 0])

    CT = 128
    pooled = pl.pallas_call(
        functools.partial(_pool_kernel, n=T, r=r),
        grid_spec=pltpu.PrefetchScalarGridSpec(
            num_scalar_prefetch=1,
            grid=(B, C // CT),
            in_specs=[
                pl.BlockSpec((1, T, CT), lambda b, c, s: (b, 0, c)),
                pl.BlockSpec((T, DP), lambda b, c, s: (0, 0)),
                pl.BlockSpec((DP, DP), lambda b, c, s: (0, 0)),
                pl.BlockSpec((1, DP), lambda b, c, s: (0, 0)),
                pl.BlockSpec((DP, DP), lambda b, c, s: (0, 0)),
            ],
            out_specs=pl.BlockSpec((1, 1, CT), lambda b, c, s: (b, 0, c)),
        ),
        out_shape=jax.ShapeDtypeStruct((B, 1, C), jnp.float32),
    )(image_lengths, val, pe, W1p, b1p, W2p)

    out = pl.pallas_call(
        _norm_kernel,
        grid=(1,),
        in_specs=[pl.BlockSpec((B, 1, C), lambda q: (0, 0, 0))],
        out_specs=pl.BlockSpec((B, 1, C), lambda q: (0, 0, 0)),
        out_shape=jax.ShapeDtypeStruct((B, 1, C), jnp.float32),
    )(pooled)
    return out.reshape(B, C)


# speedup vs baseline: 4.5240x; 1.0336x over previous
import functools

import numpy as np
import jax
import jax.numpy as jnp
from jax.experimental import pallas as pl
from jax.experimental.pallas import tpu as pltpu


NEG = -1e30


def _sinusoidal_pe(length, d):
    pos = np.arange(length, dtype=np.float32)[:, None]
    i = np.arange(d, dtype=np.float32)[None, :]
    angle = pos / np.power(10000.0, (2.0 * np.floor(i / 2.0)) / float(d))
    pe = np.where((np.arange(d)[None, :] % 2) == 0, np.sin(angle), np.cos(angle))
    return pe.astype(np.float32)


def _fused_kernel(ilen_ref, img_ref, w_ref, b_ref, pe_ref, w1_ref, b1_ref,
                  w2_ref, out_ref, vs_ref, ps_ref, *, t1, r):
    b = pl.program_id(0)
    n = 2 * t1
    x = img_ref[0]
    f = jax.lax.dot_general(x, w_ref[...], (((1,), (0,)), ((), ())),
                            preferred_element_type=jnp.float32)
    f = f + b_ref[...]
    nrm = jnp.sqrt(jnp.sum(f * f, axis=1, keepdims=True)) + 1e-8
    f = f / nrm
    ct = f.shape[1]
    f3 = f.reshape(t1, 2, ct)
    src = f3[:, 0]
    dst = f3[:, 1]
    S = jax.lax.dot_general(src, dst, (((1,), (1,)), ((), ())),
                            preferred_element_type=jnp.float32)
    nm = jnp.max(S, axis=1, keepdims=True)
    jj = jax.lax.broadcasted_iota(jnp.int32, (t1, t1), 1)
    ii = jax.lax.broadcasted_iota(jnp.int32, (t1, t1), 0)
    nidx = jnp.min(jnp.where(S == nm, jj, t1), axis=1, keepdims=True)
    nm_row = jnp.transpose(nm)
    cmp = (nm_row > nm) | ((nm_row == nm) & (jj < ii))
    rank = jnp.sum(cmp.astype(jnp.float32), axis=1, keepdims=True)
    merged = rank < float(r)
    M = jnp.where(merged & (nidx == jj), 1.0, 0.0)
    adds = jax.lax.dot_general(M, src, (((0,), (0,)), ((), ())),
                               preferred_element_type=jnp.float32)
    cnt = jax.lax.dot_general(M, jnp.ones((t1, 1), jnp.float32),
                              (((0,), (0,)), ((), ())),
                              preferred_element_type=jnp.float32)
    dst2 = (dst + adds) / (1.0 + cnt)
    vd = ilen_ref[b] - t1
    icol = jax.lax.broadcasted_iota(jnp.int32, (t1, 1), 0)
    val_src = jnp.where(merged, NEG, src)
    val_dst = jnp.where(icol < vd, dst2, NEG)
    xs = jnp.concatenate([val_src, val_dst], axis=0).astype(jnp.bfloat16)

    nt = vs_ref.shape[0]
    cw = vs_ref.shape[2]
    for t in range(nt):
        vs_ref[t] = xs[:, t * cw:(t + 1) * cw]

    i = jax.lax.broadcasted_iota(jnp.int32, (n, 1), 0)
    h = jnp.tanh(jax.lax.dot_general(pe_ref[...], w1_ref[...],
                                     (((1,), (0,)), ((), ())),
                                     preferred_element_type=jnp.float32)
                 + b1_ref[...])
    lg = jax.lax.dot_general(h, w2_ref[...], (((1,), (0,)), ((), ())),
                             preferred_element_type=jnp.float32)[:, 0:1]
    lengths = ilen_ref[b] - r
    lg = jnp.where(i < lengths, lg, -1e9)
    m = jnp.max(lg)
    e = jnp.exp(lg - m)
    w = e / jnp.sum(e)

    def tile_body(ti, carry):
        x = vs_ref[ti]
        k = 2
        while k <= n:
            j = k // 2
            while j >= 1:
                if j >= 16:
                    if k == n:
                        a2j = n // (2 * j)
                        x4 = x.reshape(a2j, 2, j, cw)
                        mx = jnp.maximum(x4[:, 0], x4[:, 1])
                        mn = jnp.minimum(x4[:, 0], x4[:, 1])
                        x = jnp.concatenate([mx[:, None], mn[:, None]],
                                            axis=1).reshape(n, cw)
                    else:
                        p = n // (2 * k)
                        q = k // (2 * j)
                        x6 = x.reshape(p, 2, q, 2, j, cw)
                        lo0 = x6[:, 0, :, 0]
                        hi0 = x6[:, 0, :, 1]
                        lo1 = x6[:, 1, :, 0]
                        hi1 = x6[:, 1, :, 1]
                        d0 = jnp.concatenate(
                            [jnp.maximum(lo0, hi0)[:, :, None],
                             jnp.minimum(lo0, hi0)[:, :, None]], axis=2)
                        d1 = jnp.concatenate(
                            [jnp.minimum(lo1, hi1)[:, :, None],
                             jnp.maximum(lo1, hi1)[:, :, None]], axis=2)
                        x = jnp.concatenate([d0[:, None], d1[:, None]],
                                            axis=1).reshape(n, cw)
                else:
                    pu = jnp.roll(x, j, axis=0)
                    pd = jnp.roll(x, -j, axis=0)
                    partner = jnp.where((i & j) != 0, pu, pd)
                    take_max = ((i & j) == 0) == ((i & k) == 0)
                    x = jnp.where(take_max, jnp.maximum(x, partner),
                                  jnp.minimum(x, partner))
                j //= 2
            k *= 2
        ps_ref[ti] = jnp.sum(x.astype(jnp.float32) * w, axis=0,
                             keepdims=True)
        return carry

    jax.lax.fori_loop(0, nt, tile_body, 0)

    pooled = jnp.concatenate([ps_ref[t] for t in range(nt)], axis=1)
    pn = jnp.sqrt(jnp.sum(pooled * pooled, axis=1, keepdims=True)) + 1e-8
    out_ref[0] = pooled / pn


@jax.jit
def kernel(images, image_lengths, W_fc, b_fc, W1, b1, W2, b2):
    B, T, K = images.shape
    C = W_fc.shape[0]
    t1 = T // 2
    r = min(T // 20, t1 // 2)
    L = T - r
    d = W1.shape[0]

    DP = 128
    pe = np.zeros((T, DP), np.float32)
    pe[:L, :d] = _sinusoidal_pe(L, d)
    pe = jnp.asarray(pe)
    W1p = jnp.zeros((DP, DP), jnp.float32).at[:d, :d].set(W1)
    b1p = jnp.zeros((1, DP), jnp.float32).at[0, :d].set(b1)
    W2p = jnp.zeros((DP, DP), jnp.float32).at[:d, 0].set(W2[:, 0])

    out = pl.pallas_call(
        functools.partial(_fused_kernel, t1=t1, r=r),
        grid_spec=pltpu.PrefetchScalarGridSpec(
            num_scalar_prefetch=1,
            grid=(B,),
            in_specs=[
                pl.BlockSpec((1, T, K), lambda b, s: (b, 0, 0)),
                pl.BlockSpec((K, C), lambda b, s: (0, 0)),
                pl.BlockSpec((1, C), lambda b, s: (0, 0)),
                pl.BlockSpec((T, DP), lambda b, s: (0, 0)),
                pl.BlockSpec((DP, DP), lambda b, s: (0, 0)),
                pl.BlockSpec((1, DP), lambda b, s: (0, 0)),
                pl.BlockSpec((DP, DP), lambda b, s: (0, 0)),
            ],
            out_specs=pl.BlockSpec((1, 1, C), lambda b, s: (b, 0, 0)),
            scratch_shapes=[
                pltpu.VMEM((C // 128, T, 128), jnp.bfloat16),
                pltpu.VMEM((C // 128, 1, 128), jnp.float32),
            ],
        ),
        out_shape=jax.ShapeDtypeStruct((B, 1, C), jnp.float32),
    )(image_lengths, images, W_fc.T, b_fc.reshape(1, C), pe, W1p, b1p, W2p)
    return out.reshape(B, C)


# speedup vs baseline: 6.6441x; 1.4686x over previous
import functools

import numpy as np
import jax
import jax.numpy as jnp
from jax.experimental import pallas as pl
from jax.experimental.pallas import tpu as pltpu


NEG = -1e30


def _sinusoidal_pe(length, d):
    pos = np.arange(length, dtype=np.float32)[:, None]
    i = np.arange(d, dtype=np.float32)[None, :]
    angle = pos / np.power(10000.0, (2.0 * np.floor(i / 2.0)) / float(d))
    pe = np.where((np.arange(d)[None, :] % 2) == 0, np.sin(angle), np.cos(angle))
    return pe.astype(np.float32)


def _proj_kernel(img_ref, w_ref, b_ref, src_ref, dst_ref):
    x = img_ref[0]
    f = jax.lax.dot_general(x, w_ref[...], (((1,), (0,)), ((), ())),
                            preferred_element_type=jnp.float32)
    f = f + b_ref[...]
    n = jnp.sqrt(jnp.sum(f * f, axis=1, keepdims=True)) + 1e-8
    f = f / n
    f3 = f.reshape(f.shape[0] // 2, 2, f.shape[1])
    src_ref[0] = f3[:, 0]
    dst_ref[0] = f3[:, 1]


def _merge_kernel(ilen_ref, src_ref, dst_ref, val_ref, *, t1, r):
    b = pl.program_id(0)
    src = src_ref[0]
    dst = dst_ref[0]
    S = jax.lax.dot_general(src, dst, (((1,), (1,)), ((), ())),
                            preferred_element_type=jnp.float32)
    nm = jnp.max(S, axis=1, keepdims=True)
    jj = jax.lax.broadcasted_iota(jnp.int32, (t1, t1), 1)
    ii = jax.lax.broadcasted_iota(jnp.int32, (t1, t1), 0)
    nidx = jnp.min(jnp.where(S == nm, jj, t1), axis=1, keepdims=True)
    nm_row = jnp.transpose(nm)
    cmp = (nm_row > nm) | ((nm_row == nm) & (jj < ii))
    rank = jnp.sum(cmp.astype(jnp.float32), axis=1, keepdims=True)
    merged = rank < float(r)
    M = jnp.where(merged & (nidx == jj), 1.0, 0.0)
    adds = jax.lax.dot_general(M, src, (((0,), (0,)), ((), ())),
                               preferred_element_type=jnp.float32)
    cnt = jax.lax.dot_general(M, jnp.ones((t1, 1), jnp.float32),
                              (((0,), (0,)), ((), ())),
                              preferred_element_type=jnp.float32)
    dst2 = (dst + adds) / (1.0 + cnt)
    vd = ilen_ref[b] - t1
    icol = jax.lax.broadcasted_iota(jnp.int32, (t1, 1), 0)
    val_src = jnp.where(merged, NEG, src)
    val_dst = jnp.where(icol < vd, dst2, NEG)
    val_ref[0] = jnp.concatenate([val_src, val_dst], axis=0)


def _pool_kernel(ilen_ref, val_ref, pe_ref, w1_ref, b1_ref, w2_ref, out_ref, *, n, r):
    b = pl.program_id(0)
    x = val_ref[0].astype(jnp.bfloat16)
    ct = x.shape[1]
    i = jax.lax.broadcasted_iota(jnp.int32, (n, 1), 0)
    k = 2
    while k <= n:
        j = k // 2
        while j >= 1:
            if j >= 16:
                if k == n:
                    a2j = n // (2 * j)
                    x4 = x.reshape(a2j, 2, j, ct)
                    mx = jnp.maximum(x4[:, 0], x4[:, 1])
                    mn = jnp.minimum(x4[:, 0], x4[:, 1])
                    x = jnp.concatenate([mx[:, None], mn[:, None]],
                                        axis=1).reshape(n, ct)
                else:
                    p = n // (2 * k)
                    q = k // (2 * j)
                    x6 = x.reshape(p, 2, q, 2, j, ct)
                    lo0 = x6[:, 0, :, 0]
                    hi0 = x6[:, 0, :, 1]
                    lo1 = x6[:, 1, :, 0]
                    hi1 = x6[:, 1, :, 1]
                    d0 = jnp.concatenate(
                        [jnp.maximum(lo0, hi0)[:, :, None],
                         jnp.minimum(lo0, hi0)[:, :, None]], axis=2)
                    d1 = jnp.concatenate(
                        [jnp.minimum(lo1, hi1)[:, :, None],
                         jnp.maximum(lo1, hi1)[:, :, None]], axis=2)
                    x = jnp.concatenate([d0[:, None], d1[:, None]],
                                        axis=1).reshape(n, ct)
            else:
                pu = jnp.roll(x, j, axis=0)
                pd = jnp.roll(x, -j, axis=0)
                partner = jnp.where((i & j) != 0, pu, pd)
                take_max = ((i & j) == 0) == ((i & k) == 0)
                x = jnp.where(take_max, jnp.maximum(x, partner),
                              jnp.minimum(x, partner))
            j //= 2
        k *= 2
    h = jnp.tanh(jax.lax.dot_general(pe_ref[...], w1_ref[...],
                                     (((1,), (0,)), ((), ())),
                                     preferred_element_type=jnp.float32)
                 + b1_ref[...])
    lg = jax.lax.dot_general(h, w2_ref[...], (((1,), (0,)), ((), ())),
                             preferred_element_type=jnp.float32)[:, 0:1]
    lengths = ilen_ref[b] - r
    lg = jnp.where(i < lengths, lg, -1e9)
    m = jnp.max(lg)
    e = jnp.exp(lg - m)
    w = e / jnp.sum(e)
    pooled = jnp.sum(x.astype(jnp.float32) * w, axis=0,
                     keepdims=True)
    out_ref[0] = pooled


def _norm_kernel(p_ref, out_ref):
    p = p_ref[...]
    nrm = jnp.sqrt(jnp.sum(p * p, axis=2, keepdims=True)) + 1e-8
    out_ref[...] = p / nrm


@jax.jit
def kernel(images, image_lengths, W_fc, b_fc, W1, b1, W2, b2):
    B, T, K = images.shape
    C = W_fc.shape[0]
    t1 = T // 2
    r = min(T // 20, t1 // 2)
    L = T - r
    d = W1.shape[0]

    RT = min(256, T)
    srcN, dstN = pl.pallas_call(
        _proj_kernel,
        grid=(B, T // RT),
        in_specs=[
            pl.BlockSpec((1, RT, K), lambda b, t: (b, t, 0)),
            pl.BlockSpec((K, C), lambda b, t: (0, 0)),
            pl.BlockSpec((1, C), lambda b, t: (0, 0)),
        ],
        out_specs=[
            pl.BlockSpec((1, RT // 2, C), lambda b, t: (b, t, 0)),
            pl.BlockSpec((1, RT // 2, C), lambda b, t: (b, t, 0)),
        ],
        out_shape=[
            jax.ShapeDtypeStruct((B, t1, C), jnp.float32),
            jax.ShapeDtypeStruct((B, t1, C), jnp.float32),
        ],
    )(images, W_fc.T, b_fc.reshape(1, C))

    val = pl.pallas_call(
        functools.partial(_merge_kernel, t1=t1, r=r),
        grid_spec=pltpu.PrefetchScalarGridSpec(
            num_scalar_prefetch=1,
            grid=(B,),
            in_specs=[
                pl.BlockSpec((1, t1, C), lambda b, s: (b, 0, 0)),
                pl.BlockSpec((1, t1, C), lambda b, s: (b, 0, 0)),
            ],
            out_specs=pl.BlockSpec((1, T, C), lambda b, s: (b, 0, 0)),
        ),
        out_shape=jax.ShapeDtypeStruct((B, T, C), jnp.float32),
    )(image_lengths, srcN, dstN)

    DP = 128
    pe = np.zeros((T, DP), np.float32)
    pe[:L, :d] = _sinusoidal_pe(L, d)
    W1p = np.zeros((DP, DP), np.float32)
    b1p = np.zeros((1, DP), np.float32)
    W2p = np.zeros((DP, DP), np.float32)
    pe = jnp.asarray(pe)
    W1p = jnp.asarray(W1p).at[:d, :d].set(W1)
    b1p = jnp.asarray(b1p).at[0, :d].set(b1)
    W2p = jnp.asarray(W2p).at[:d, 0].set(W2[:, 0])

    CT = 128
    pooled = pl.pallas_call(
        functools.partial(_pool_kernel, n=T, r=r),
        grid_spec=pltpu.PrefetchScalarGridSpec(
            num_scalar_prefetch=1,
            grid=(B, C // CT),
            in_specs=[
                pl.BlockSpec((1, T, CT), lambda b, c, s: (b, 0, c)),
                pl.BlockSpec((T, DP), lambda b, c, s: (0, 0)),
                pl.BlockSpec((DP, DP), lambda b, c, s: (0, 0)),
                pl.BlockSpec((1, DP), lambda b, c, s: (0, 0)),
                pl.BlockSpec((DP, DP), lambda b, c, s: (0, 0)),
            ],
            out_specs=pl.BlockSpec((1, 1, CT), lambda b, c, s: (b, 0, c)),
        ),
        out_shape=jax.ShapeDtypeStruct((B, 1, C), jnp.float32),
    )(image_lengths, val, pe, W1p, b1p, W2p)

    out = pl.pallas_call(
        _norm_kernel,
        grid=(1,),
        in_specs=[pl.BlockSpec((B, 1, C), lambda q: (0, 0, 0))],
        out_specs=pl.BlockSpec((B, 1, C), lambda q: (0, 0, 0)),
        out_shape=jax.ShapeDtypeStruct((B, 1, C), jnp.float32),
    )(pooled)
    return out.reshape(B, C)


# speedup vs baseline: 6.8450x; 1.0302x over previous
import functools

import numpy as np
import jax
import jax.numpy as jnp
from jax.experimental import pallas as pl
from jax.experimental.pallas import tpu as pltpu


NEG = -1e30


def _sinusoidal_pe(length, d):
    pos = np.arange(length, dtype=np.float32)[:, None]
    i = np.arange(d, dtype=np.float32)[None, :]
    angle = pos / np.power(10000.0, (2.0 * np.floor(i / 2.0)) / float(d))
    pe = np.where((np.arange(d)[None, :] % 2) == 0, np.sin(angle), np.cos(angle))
    return pe.astype(np.float32)


def _proj_kernel(img_ref, w_ref, b_ref, src_ref, dst_ref):
    x = img_ref[0]
    f = jax.lax.dot_general(x, w_ref[...], (((1,), (0,)), ((), ())),
                            preferred_element_type=jnp.float32)
    f = f + b_ref[...]
    n = jnp.sqrt(jnp.sum(f * f, axis=1, keepdims=True)) + 1e-8
    f = f / n
    f3 = f.reshape(f.shape[0] // 2, 2, f.shape[1])
    src_ref[0] = f3[:, 0]
    dst_ref[0] = f3[:, 1]


def _merge_kernel(ilen_ref, src_ref, dst_ref, val_ref, *, t1, r):
    b = pl.program_id(0)
    src = src_ref[0]
    dst = dst_ref[0]
    S = jax.lax.dot_general(src, dst, (((1,), (1,)), ((), ())),
                            preferred_element_type=jnp.float32)
    nm = jnp.max(S, axis=1, keepdims=True)
    jj = jax.lax.broadcasted_iota(jnp.int32, (t1, t1), 1)
    ii = jax.lax.broadcasted_iota(jnp.int32, (t1, t1), 0)
    nidx = jnp.min(jnp.where(S == nm, jj, t1), axis=1, keepdims=True)
    nm_row = jnp.transpose(nm)
    cmp = (nm_row > nm) | ((nm_row == nm) & (jj < ii))
    rank = jnp.sum(cmp.astype(jnp.float32), axis=1, keepdims=True)
    merged = rank < float(r)
    M = jnp.where(merged & (nidx == jj), 1.0, 0.0)
    adds = jax.lax.dot_general(M, src, (((0,), (0,)), ((), ())),
                               preferred_element_type=jnp.float32)
    cnt = jax.lax.dot_general(M, jnp.ones((t1, 1), jnp.float32),
                              (((0,), (0,)), ((), ())),
                              preferred_element_type=jnp.float32)
    dst2 = (dst + adds) / (1.0 + cnt)
    vd = ilen_ref[b] - t1
    icol = jax.lax.broadcasted_iota(jnp.int32, (t1, 1), 0)
    val_src = jnp.where(merged, NEG, src)
    val_dst = jnp.where(icol < vd, dst2, NEG)
    val_ref[0] = jnp.concatenate([val_src, val_dst], axis=0)


def _pool_kernel(ilen_ref, val_ref, pe_ref, w1_ref, b1_ref, w2_ref, out_ref, *, n, r):
    b = pl.program_id(0)
    x = val_ref[0].astype(jnp.bfloat16)
    ct = x.shape[1]
    i = jax.lax.broadcasted_iota(jnp.int32, (n, 1), 0)
    k = 2
    while k <= n:
        j = k // 2
        while j >= 1:
            if j >= 16:
                if k == n:
                    a2j = n // (2 * j)
                    x4 = x.reshape(a2j, 2, j, ct)
                    mx = jnp.maximum(x4[:, 0], x4[:, 1])
                    mn = jnp.minimum(x4[:, 0], x4[:, 1])
                    x = jnp.concatenate([mx[:, None], mn[:, None]],
                                        axis=1).reshape(n, ct)
                else:
                    p = n // (2 * k)
                    q = k // (2 * j)
                    x6 = x.reshape(p, 2, q, 2, j, ct)
                    lo0 = x6[:, 0, :, 0]
                    hi0 = x6[:, 0, :, 1]
                    lo1 = x6[:, 1, :, 0]
                    hi1 = x6[:, 1, :, 1]
                    d0 = jnp.concatenate(
                        [jnp.maximum(lo0, hi0)[:, :, None],
                         jnp.minimum(lo0, hi0)[:, :, None]], axis=2)
                    d1 = jnp.concatenate(
                        [jnp.minimum(lo1, hi1)[:, :, None],
                         jnp.maximum(lo1, hi1)[:, :, None]], axis=2)
                    x = jnp.concatenate([d0[:, None], d1[:, None]],
                                        axis=1).reshape(n, ct)
            elif k >= 16:
                if k == n:
                    is_lo = (i & j) == 0
                    x = jnp.where(is_lo,
                                  jnp.maximum(x, jnp.roll(x, -j, axis=0)),
                                  jnp.minimum(x, jnp.roll(x, j, axis=0)))
                    j //= 2
                    continue
                p = n // (2 * k)
                x3 = x.reshape(p, 2, k, ct)
                d0 = x3[:, 0]
                d1 = x3[:, 1]
                ik = jax.lax.broadcasted_iota(jnp.int32, (1, k, 1), 1)
                is_lo = (ik & j) == 0
                n0 = jnp.where(is_lo,
                               jnp.maximum(d0, jnp.roll(d0, -j, axis=1)),
                               jnp.minimum(d0, jnp.roll(d0, j, axis=1)))
                n1 = jnp.where(is_lo,
                               jnp.minimum(d1, jnp.roll(d1, -j, axis=1)),
                               jnp.maximum(d1, jnp.roll(d1, j, axis=1)))
                x = jnp.concatenate([n0[:, None], n1[:, None]],
                                    axis=1).reshape(n, ct)
            else:
                pu = jnp.roll(x, j, axis=0)
                pd = jnp.roll(x, -j, axis=0)
                partner = jnp.where((i & j) != 0, pu, pd)
                take_max = ((i & j) == 0) == ((i & k) == 0)
                x = jnp.where(take_max, jnp.maximum(x, partner),
                              jnp.minimum(x, partner))
            j //= 2
        k *= 2
    h = jnp.tanh(jax.lax.dot_general(pe_ref[...], w1_ref[...],
                                     (((1,), (0,)), ((), ())),
                                     preferred_element_type=jnp.float32)
                 + b1_ref[...])
    lg = jax.lax.dot_general(h, w2_ref[...], (((1,), (0,)), ((), ())),
                             preferred_element_type=jnp.float32)[:, 0:1]
    lengths = ilen_ref[b] - r
    lg = jnp.where(i < lengths, lg, -1e9)
    m = jnp.max(lg)
    e = jnp.exp(lg - m)
    w = e / jnp.sum(e)
    pooled = jnp.sum(x.astype(jnp.float32) * w, axis=0,
                     keepdims=True)
    out_ref[0] = pooled


def _norm_kernel(p_ref, out_ref):
    p = p_ref[...]
    nrm = jnp.sqrt(jnp.sum(p * p, axis=2, keepdims=True)) + 1e-8
    out_ref[...] = p / nrm


@jax.jit
def kernel(images, image_lengths, W_fc, b_fc, W1, b1, W2, b2):
    B, T, K = images.shape
    C = W_fc.shape[0]
    t1 = T // 2
    r = min(T // 20, t1 // 2)
    L = T - r
    d = W1.shape[0]

    RT = min(256, T)
    srcN, dstN = pl.pallas_call(
        _proj_kernel,
        grid=(B, T // RT),
        in_specs=[
            pl.BlockSpec((1, RT, K), lambda b, t: (b, t, 0)),
            pl.BlockSpec((K, C), lambda b, t: (0, 0)),
            pl.BlockSpec((1, C), lambda b, t: (0, 0)),
        ],
        out_specs=[
            pl.BlockSpec((1, RT // 2, C), lambda b, t: (b, t, 0)),
            pl.BlockSpec((1, RT // 2, C), lambda b, t: (b, t, 0)),
        ],
        out_shape=[
            jax.ShapeDtypeStruct((B, t1, C), jnp.float32),
            jax.ShapeDtypeStruct((B, t1, C), jnp.float32),
        ],
    )(images, W_fc.T, b_fc.reshape(1, C))

    val = pl.pallas_call(
        functools.partial(_merge_kernel, t1=t1, r=r),
        grid_spec=pltpu.PrefetchScalarGridSpec(
            num_scalar_prefetch=1,
            grid=(B,),
            in_specs=[
                pl.BlockSpec((1, t1, C), lambda b, s: (b, 0, 0)),
                pl.BlockSpec((1, t1, C), lambda b, s: (b, 0, 0)),
            ],
            out_specs=pl.BlockSpec((1, T, C), lambda b, s: (b, 0, 0)),
        ),
        out_shape=jax.ShapeDtypeStruct((B, T, C), jnp.float32),
    )(image_lengths, srcN, dstN)

    DP = 128
    pe = np.zeros((T, DP), np.float32)
    pe[:L, :d] = _sinusoidal_pe(L, d)
    W1p = np.zeros((DP, DP), np.float32)
    b1p = np.zeros((1, DP), np.float32)
    W2p = np.zeros((DP, DP), np.float32)
    pe = jnp.asarray(pe)
    W1p = jnp.asarray(W1p).at[:d, :d].set(W1)
    b1p = jnp.asarray(b1p).at[0, :d].set(b1)
    W2p = jnp.asarray(W2p).at[:d, 0].set(W2[:, 0])

    CT = 128
    pooled = pl.pallas_call(
        functools.partial(_pool_kernel, n=T, r=r),
        grid_spec=pltpu.PrefetchScalarGridSpec(
            num_scalar_prefetch=1,
            grid=(B, C // CT),
            in_specs=[
                pl.BlockSpec((1, T, CT), lambda b, c, s: (b, 0, c)),
                pl.BlockSpec((T, DP), lambda b, c, s: (0, 0)),
                pl.BlockSpec((DP, DP), lambda b, c, s: (0, 0)),
                pl.BlockSpec((1, DP), lambda b, c, s: (0, 0)),
                pl.BlockSpec((DP, DP), lambda b, c, s: (0, 0)),
            ],
            out_specs=pl.BlockSpec((1, 1, CT), lambda b, c, s: (b, 0, c)),
        ),
        out_shape=jax.ShapeDtypeStruct((B, 1, C), jnp.float32),
    )(image_lengths, val, pe, W1p, b1p, W2p)

    out = pl.pallas_call(
        _norm_kernel,
        grid=(1,),
        in_specs=[pl.BlockSpec((B, 1, C), lambda q: (0, 0, 0))],
        out_specs=pl.BlockSpec((B, 1, C), lambda q: (0, 0, 0)),
        out_shape=jax.ShapeDtypeStruct((B, 1, C), jnp.float32),
    )(pooled)
    return out.reshape(B, C)


# speedup vs baseline: 7.1079x; 1.0384x over previous
import functools

import numpy as np
import jax
import jax.numpy as jnp
from jax.experimental import pallas as pl
from jax.experimental.pallas import tpu as pltpu


NEG = -1e30


def _sinusoidal_pe(length, d):
    pos = np.arange(length, dtype=np.float32)[:, None]
    i = np.arange(d, dtype=np.float32)[None, :]
    angle = pos / np.power(10000.0, (2.0 * np.floor(i / 2.0)) / float(d))
    pe = np.where((np.arange(d)[None, :] % 2) == 0, np.sin(angle), np.cos(angle))
    return pe.astype(np.float32)


def _proj_kernel(img_ref, w_ref, b_ref, src_ref, dst_ref):
    x = img_ref[0]
    f = jax.lax.dot_general(x, w_ref[...], (((1,), (0,)), ((), ())),
                            preferred_element_type=jnp.float32)
    f = f + b_ref[...]
    n = jnp.sqrt(jnp.sum(f * f, axis=1, keepdims=True)) + 1e-8
    f = f / n
    f3 = f.reshape(f.shape[0] // 2, 2, f.shape[1])
    src_ref[0] = f3[:, 0]
    dst_ref[0] = f3[:, 1]


def _merge_kernel(ilen_ref, src_ref, dst_ref, val_ref, *, t1, r):
    b = pl.program_id(0)
    src = src_ref[0]
    dst = dst_ref[0]
    S = jax.lax.dot_general(src, dst, (((1,), (1,)), ((), ())),
                            preferred_element_type=jnp.float32)
    nm = jnp.max(S, axis=1, keepdims=True)
    jj = jax.lax.broadcasted_iota(jnp.int32, (t1, t1), 1)
    ii = jax.lax.broadcasted_iota(jnp.int32, (t1, t1), 0)
    nidx = jnp.min(jnp.where(S == nm, jj, t1), axis=1, keepdims=True)
    nm_row = jnp.transpose(nm)
    cmp = (nm_row > nm) | ((nm_row == nm) & (jj < ii))
    rank = jnp.sum(cmp.astype(jnp.float32), axis=1, keepdims=True)
    merged = rank < float(r)
    M = jnp.where(merged & (nidx == jj), 1.0, 0.0)
    adds = jax.lax.dot_general(M, src, (((0,), (0,)), ((), ())),
                               preferred_element_type=jnp.float32)
    cnt = jax.lax.dot_general(M, jnp.ones((t1, 1), jnp.float32),
                              (((0,), (0,)), ((), ())),
                              preferred_element_type=jnp.float32)
    dst2 = (dst + adds) / (1.0 + cnt)
    vd = ilen_ref[b] - t1
    icol = jax.lax.broadcasted_iota(jnp.int32, (t1, 1), 0)
    val_src = jnp.where(merged, NEG, src)
    val_dst = jnp.where(icol < vd, dst2, NEG)
    val_ref[0] = jnp.concatenate([val_src, val_dst], axis=0)


def _pool_kernel(ilen_ref, val_ref, pe_ref, w1_ref, b1_ref, w2_ref, out_ref, *, n, r):
    b = pl.program_id(0)
    x = val_ref[0].astype(jnp.bfloat16)
    ct = x.shape[1]
    i = jax.lax.broadcasted_iota(jnp.int32, (n, 1), 0)
    k = 2
    while k <= n:
        j = k // 2
        while j >= 1:
            if j >= 16:
                if k == n:
                    a2j = n // (2 * j)
                    x4 = x.reshape(a2j, 2, j, ct)
                    mx = jnp.maximum(x4[:, 0], x4[:, 1])
                    mn = jnp.minimum(x4[:, 0], x4[:, 1])
                    x = jnp.concatenate([mx[:, None], mn[:, None]],
                                        axis=1).reshape(n, ct)
                else:
                    p = n // (2 * k)
                    q = k // (2 * j)
                    x6 = x.reshape(p, 2, q, 2, j, ct)
                    lo0 = x6[:, 0, :, 0]
                    hi0 = x6[:, 0, :, 1]
                    lo1 = x6[:, 1, :, 0]
                    hi1 = x6[:, 1, :, 1]
                    d0 = jnp.concatenate(
                        [jnp.maximum(lo0, hi0)[:, :, None],
                         jnp.minimum(lo0, hi0)[:, :, None]], axis=2)
                    d1 = jnp.concatenate(
                        [jnp.minimum(lo1, hi1)[:, :, None],
                         jnp.maximum(lo1, hi1)[:, :, None]], axis=2)
                    x = jnp.concatenate([d0[:, None], d1[:, None]],
                                        axis=1).reshape(n, ct)
            elif k >= 16:
                if k == n:
                    is_lo = (i & j) == 0
                    x = jnp.where(is_lo,
                                  jnp.maximum(x, jnp.roll(x, -j, axis=0)),
                                  jnp.minimum(x, jnp.roll(x, j, axis=0)))
                    j //= 2
                    continue
                p = n // (2 * k)
                x3 = x.reshape(p, 2, k, ct)
                d0 = x3[:, 0]
                d1 = x3[:, 1]
                ik = jax.lax.broadcasted_iota(jnp.int32, (1, k, 1), 1)
                is_lo = (ik & j) == 0
                n0 = jnp.where(is_lo,
                               jnp.maximum(d0, jnp.roll(d0, -j, axis=1)),
                               jnp.minimum(d0, jnp.roll(d0, j, axis=1)))
                n1 = jnp.where(is_lo,
                               jnp.minimum(d1, jnp.roll(d1, -j, axis=1)),
                               jnp.maximum(d1, jnp.roll(d1, j, axis=1)))
                x = jnp.concatenate([n0[:, None], n1[:, None]],
                                    axis=1).reshape(n, ct)
            else:
                pu = jnp.roll(x, j, axis=0)
                pd = jnp.roll(x, -j, axis=0)
                partner = jnp.where((i & j) != 0, pu, pd)
                take_max = ((i & j) == 0) == ((i & k) == 0)
                x = jnp.where(take_max, jnp.maximum(x, partner),
                              jnp.minimum(x, partner))
            j //= 2
        k *= 2
    h = jnp.tanh(jax.lax.dot_general(pe_ref[...], w1_ref[...],
                                     (((1,), (0,)), ((), ())),
                                     preferred_element_type=jnp.float32)
                 + b1_ref[...])
    lg = jax.lax.dot_general(h, w2_ref[...], (((1,), (0,)), ((), ())),
                             preferred_element_type=jnp.float32)[:, 0:1]
    lengths = ilen_ref[b] - r
    lg = jnp.where(i < lengths, lg, -1e9)
    m = jnp.max(lg)
    e = jnp.exp(lg - m)
    w = e / jnp.sum(e)
    pooled = jnp.sum(x.astype(jnp.float32) * w, axis=0,
                     keepdims=True)
    out_ref[0] = pooled


def _norm_kernel(p_ref, out_ref):
    p = p_ref[...]
    nrm = jnp.sqrt(jnp.sum(p * p, axis=2, keepdims=True)) + 1e-8
    out_ref[...] = p / nrm


@jax.jit
def kernel(images, image_lengths, W_fc, b_fc, W1, b1, W2, b2):
    B, T, K = images.shape
    C = W_fc.shape[0]
    t1 = T // 2
    r = min(T // 20, t1 // 2)
    L = T - r
    d = W1.shape[0]

    RT = min(512, T)
    srcN, dstN = pl.pallas_call(
        _proj_kernel,
        grid=(B, T // RT),
        in_specs=[
            pl.BlockSpec((1, RT, K), lambda b, t: (b, t, 0)),
            pl.BlockSpec((K, C), lambda b, t: (0, 0)),
            pl.BlockSpec((1, C), lambda b, t: (0, 0)),
        ],
        out_specs=[
            pl.BlockSpec((1, RT // 2, C), lambda b, t: (b, t, 0)),
            pl.BlockSpec((1, RT // 2, C), lambda b, t: (b, t, 0)),
        ],
        out_shape=[
            jax.ShapeDtypeStruct((B, t1, C), jnp.float32),
            jax.ShapeDtypeStruct((B, t1, C), jnp.float32),
        ],
    )(images, W_fc.T, b_fc.reshape(1, C))

    val = pl.pallas_call(
        functools.partial(_merge_kernel, t1=t1, r=r),
        grid_spec=pltpu.PrefetchScalarGridSpec(
            num_scalar_prefetch=1,
            grid=(B,),
            in_specs=[
                pl.BlockSpec((1, t1, C), lambda b, s: (b, 0, 0)),
                pl.BlockSpec((1, t1, C), lambda b, s: (b, 0, 0)),
            ],
            out_specs=pl.BlockSpec((1, T, C), lambda b, s: (b, 0, 0)),
        ),
        out_shape=jax.ShapeDtypeStruct((B, T, C), jnp.float32),
    )(image_lengths, srcN, dstN)

    DP = 128
    pe = np.zeros((T, DP), np.float32)
    pe[:L, :d] = _sinusoidal_pe(L, d)
    W1p = np.zeros((DP, DP), np.float32)
    b1p = np.zeros((1, DP), np.float32)
    W2p = np.zeros((DP, DP), np.float32)
    pe = jnp.asarray(pe)
    W1p = jnp.asarray(W1p).at[:d, :d].set(W1)
    b1p = jnp.asarray(b1p).at[0, :d].set(b1)
    W2p = jnp.asarray(W2p).at[:d, 0].set(W2[:, 0])

    CT = 256
    pooled = pl.pallas_call(
        functools.partial(_pool_kernel, n=T, r=r),
        grid_spec=pltpu.PrefetchScalarGridSpec(
            num_scalar_prefetch=1,
            grid=(B, C // CT),
            in_specs=[
                pl.BlockSpec((1, T, CT), lambda b, c, s: (b, 0, c)),
                pl.BlockSpec((T, DP), lambda b, c, s: (0, 0)),
                pl.BlockSpec((DP, DP), lambda b, c, s: (0, 0)),
                pl.BlockSpec((1, DP), lambda b, c, s: (0, 0)),
                pl.BlockSpec((DP, DP), lambda b, c, s: (0, 0)),
            ],
            out_specs=pl.BlockSpec((1, 1, CT), lambda b, c, s: (b, 0, c)),
        ),
        out_shape=jax.ShapeDtypeStruct((B, 1, C), jnp.float32),
    )(image_lengths, val, pe, W1p, b1p, W2p)

    out = pl.pallas_call(
        _norm_kernel,
        grid=(1,),
        in_specs=[pl.BlockSpec((B, 1, C), lambda q: (0, 0, 0))],
        out_specs=pl.BlockSpec((B, 1, C), lambda q: (0, 0, 0)),
        out_shape=jax.ShapeDtypeStruct((B, 1, C), jnp.float32),
    )(pooled)
    return out.reshape(B, C)
